# Initial kernel scaffold; baseline (speedup 1.0000x reference)
import jax
import jax.numpy as jnp
from jax.experimental import pallas as pl


def kernel(x, edge_index, batch, edge_weight, conv1_W, conv1_b, conv2_W, conv2_b, conv1d_W, conv1d_b, fc_W, fc_b):
    raise NotImplementedError("write your pallas kernel here")


# speedup vs baseline: 1.9940x; 1.9940x over previous
import functools

import jax
import jax.numpy as jnp
from jax import lax
from jax.experimental import pallas as pl
from jax.experimental.pallas import tpu as pltpu
from jax.experimental.pallas import tpu_sc as plsc

N = 10000
E = 322560
G = 100
KP = 12
C = 4
D = 128
NEW = 4032
NC = 2
NS = 16
NW = NC * NS
NP = 10240
NF = NP * C
RT = NP // NW
EMAX = 12288
CHB = 5040
CHE = 128

_F = jnp.float32
_I = jnp.int32


def _mesh():
    return plsc.VectorSubcoreMesh(
        core_axis_name="c", subcore_axis_name="s",
        num_cores=NC, num_subcores=NS)


def _wid():
    return lax.axis_index("s") * NC + lax.axis_index("c")


def _zero_ref(ref, nwords):
    z = jnp.zeros((16,), _F)

    def body(i, _):
        ref[pl.ds(i * 16, 16)] = z
        return 0
    lax.fori_loop(0, nwords // 16, body, 0)


def _sc_bin_body(row_hbm, col_hbm, ewt_hbm, brow_hbm, bcol_hbm, bew_hbm,
                 deg_hbm, rbuf, cbuf, ewt, lrow, lcol, lew, dacc):
    wid = _wid()
    base_n = wid * RT
    iota = lax.iota(_I, 16)
    pltpu.sync_copy(ewt_hbm, ewt)
    _zero_ref(dacc, RT)
    _zero_ref(lew, EMAX)
    zi = jnp.zeros((16,), _I)

    def zl(i, _):
        lrow[pl.ds(i * 16, 16)] = zi
        lcol[pl.ds(i * 16, 16)] = zi
        return 0
    lax.fori_loop(0, EMAX // 16, zl, 0)

    def chunk(k, off):
        eb = k * CHB
        pltpu.sync_copy(row_hbm.at[pl.ds(eb, CHB)], rbuf)
        pltpu.sync_copy(col_hbm.at[pl.ds(eb, CHB)], cbuf)

        def grp(g, off):
            sl = pl.ds(g * 16, 16)
            r16 = rbuf[sl]
            c16 = cbuf[sl]
            e16 = eb + g * 16 + iota
            m16 = e16 % NEW
            ewv = plsc.load_gather(ewt, [m16])
            loc = c16 - base_n
            sel = jnp.logical_and(loc >= 0, loc < RT)
            locc = jnp.minimum(jnp.maximum(loc, 0), RT - 1)
            cnt = plsc.all_reduce_population_count(sel)[0]

            @pl.when(off <= EMAX - 16)
            def _():
                plsc.addupdate_scatter(dacc, [locc], ewv, mask=sel)
                plsc.store_compressed(lrow.at[pl.ds(off, 16)], r16, mask=sel)
                plsc.store_compressed(lcol.at[pl.ds(off, 16)], c16, mask=sel)
                plsc.store_compressed(lew.at[pl.ds(off, 16)], ewv, mask=sel)
            return off + cnt
        return lax.fori_loop(0, CHB // 16, grp, off)

    lax.fori_loop(0, E // CHB, chunk, jnp.int32(0))
    base_e = wid * EMAX
    pltpu.sync_copy(lrow, brow_hbm.at[pl.ds(base_e, EMAX)])
    pltpu.sync_copy(lcol, bcol_hbm.at[pl.ds(base_e, EMAX)])
    pltpu.sync_copy(lew, bew_hbm.at[pl.ds(base_e, EMAX)])
    pltpu.sync_copy(dacc, deg_hbm.at[pl.ds(base_n, RT)])


def _make_sc_bin():
    return pl.kernel(
        _sc_bin_body,
        out_type=(jax.ShapeDtypeStruct((NW * EMAX,), _I),
                  jax.ShapeDtypeStruct((NW * EMAX,), _I),
                  jax.ShapeDtypeStruct((NW * EMAX,), _F),
                  jax.ShapeDtypeStruct((NP,), _F)),
        mesh=_mesh(),
        compiler_params=pltpu.CompilerParams(needs_layout_passes=False),
        scratch_types=[
            pltpu.VMEM((CHB,), _I),
            pltpu.VMEM((CHB,), _I),
            pltpu.VMEM((NEW,), _F),
            pltpu.VMEM((EMAX,), _I),
            pltpu.VMEM((EMAX,), _I),
            pltpu.VMEM((EMAX,), _F),
            pltpu.VMEM((RT,), _F),
        ])


def _sc_normb_body(brow_hbm, bcol_hbm, bew_hbm, dinv_hbm, bnorm_hbm,
                   dv, rbuf, cbuf, wbuf, nbuf):
    wid = _wid()
    base_e = wid * EMAX
    pltpu.sync_copy(dinv_hbm, dv)
    pltpu.sync_copy(brow_hbm.at[pl.ds(base_e, EMAX)], rbuf)
    pltpu.sync_copy(bcol_hbm.at[pl.ds(base_e, EMAX)], cbuf)
    pltpu.sync_copy(bew_hbm.at[pl.ds(base_e, EMAX)], wbuf)

    def body(g, _):
        sl = pl.ds(g * 16, 16)
        a = plsc.load_gather(dv, [rbuf[sl]])
        b = plsc.load_gather(dv, [cbuf[sl]])
        nbuf[sl] = a * wbuf[sl] * b
        return 0
    lax.fori_loop(0, EMAX // 16, body, 0)
    pltpu.sync_copy(nbuf, bnorm_hbm.at[pl.ds(base_e, EMAX)])


def _make_sc_normb():
    return pl.kernel(
        _sc_normb_body,
        out_type=jax.ShapeDtypeStruct((NW * EMAX,), _F),
        mesh=_mesh(),
        compiler_params=pltpu.CompilerParams(needs_layout_passes=False),
        scratch_types=[
            pltpu.VMEM((NP,), _F),
            pltpu.VMEM((EMAX,), _I),
            pltpu.VMEM((EMAX,), _I),
            pltpu.VMEM((EMAX,), _F),
            pltpu.VMEM((EMAX,), _F),
        ])


def _sc_hop128_body(zh_hbm, brow_hbm, bnorm_hbm, bcol_hbm, out_hbm,
                    acc, gbuf, rbuf, cbuf, nbuf, sem):
    wid = _wid()
    base_n = wid * RT
    base_e = wid * EMAX
    pltpu.sync_copy(brow_hbm.at[pl.ds(base_e, EMAX)], rbuf)
    pltpu.sync_copy(bcol_hbm.at[pl.ds(base_e, EMAX)], cbuf)
    pltpu.sync_copy(bnorm_hbm.at[pl.ds(base_e, EMAX)], nbuf)

    def zrow(i, _):
        for c in range(D // 16):
            acc[i, pl.ds(c * 16, 16)] = jnp.zeros((16,), _F)
        return 0
    lax.fori_loop(0, RT, zrow, 0)

    def chunk(k, _):
        cb = k * CHE
        pltpu.async_copy(zh_hbm.at[rbuf.at[pl.ds(cb, CHE)]], gbuf,
                         sem).wait()

        def edge(e, _):
            nrm = nbuf[pl.ds(cb + e, 16)][0]
            loc = cbuf[pl.ds(cb + e, 16)][0] - base_n
            loc = jnp.minimum(jnp.maximum(loc, 0), RT - 1)
            nv = jnp.full((16,), nrm, _F)
            for c in range(D // 16):
                s = pl.ds(c * 16, 16)
                acc[loc, s] = acc[loc, s] + gbuf[e, s] * nv
            return 0
        lax.fori_loop(0, CHE, edge, 0)
        return 0
    lax.fori_loop(0, EMAX // CHE, chunk, 0)
    pltpu.sync_copy(acc, out_hbm.at[pl.ds(base_n, RT)])


def _make_sc_hop128():
    return pl.kernel(
        _sc_hop128_body,
        out_type=jax.ShapeDtypeStruct((NP, D), _F),
        mesh=_mesh(),
        compiler_params=pltpu.CompilerParams(needs_layout_passes=False),
        scratch_types=[
            pltpu.VMEM((RT, D), _F),
            pltpu.VMEM((CHE, D), _F),
            pltpu.VMEM((EMAX,), _I),
            pltpu.VMEM((EMAX,), _I),
            pltpu.VMEM((EMAX,), _F),
            pltpu.SemaphoreType.DMA,
        ])


def _sc_hop4_body(zh_hbm, brow_hbm, bnorm_hbm, bcol_hbm, out_hbm,
                  zv, acc, rbuf, cbuf, nbuf):
    wid = _wid()
    base_n = wid * RT
    base_e = wid * EMAX
    pltpu.sync_copy(zh_hbm, zv)
    pltpu.sync_copy(brow_hbm.at[pl.ds(base_e, EMAX)], rbuf)
    pltpu.sync_copy(bcol_hbm.at[pl.ds(base_e, EMAX)], cbuf)
    pltpu.sync_copy(bnorm_hbm.at[pl.ds(base_e, EMAX)], nbuf)
    _zero_ref(acc, RT * C)

    def body(g, _):
        sl = pl.ds(g * 16, 16)
        r16 = rbuf[sl] * C
        n16 = nbuf[sl]
        loc = cbuf[sl] - base_n
        loc = jnp.minimum(jnp.maximum(loc, 0), RT - 1) * C
        for c in range(C):
            v = plsc.load_gather(zv, [r16 + c]) * n16
            plsc.addupdate_scatter(acc, [loc + c], v)
        return 0
    lax.fori_loop(0, EMAX // 16, body, 0)
    pltpu.sync_copy(acc, out_hbm.at[pl.ds(base_n * C, RT * C)])


def _make_sc_hop4():
    return pl.kernel(
        _sc_hop4_body,
        out_type=jax.ShapeDtypeStruct((NF,), _F),
        mesh=_mesh(),
        compiler_params=pltpu.CompilerParams(needs_layout_passes=False),
        scratch_types=[
            pltpu.VMEM((NF,), _F),
            pltpu.VMEM((RT * C,), _F),
            pltpu.VMEM((EMAX,), _I),
            pltpu.VMEM((EMAX,), _I),
            pltpu.VMEM((EMAX,), _F),
        ])


def _sc_pool_body(h2_hbm, starts_hbm, counts_hbm, pool_hbm, hv, stb, ctb,
                  prow):
    wid = _wid()
    pltpu.sync_copy(h2_hbm, hv)
    pltpu.sync_copy(starts_hbm, stb.at[pl.ds(0, G)])
    pltpu.sync_copy(counts_hbm, ctb.at[pl.ds(0, G)])
    iota = lax.iota(_I, 16)
    zf = jnp.zeros((16,), _F)
    one = jnp.ones((16,), _I)
    zi = jnp.zeros((16,), _I)
    wmask = iota < C

    for j in range((G + NW - 1) // NW):
        g = wid + NW * j

        @pl.when(g < G)
        def _():
            st = stb[pl.ds(g, 16)][0]
            n = ctb[pl.ds(g, 16)][0]
            prow[pl.ds(0, 16)] = zf
            prow[pl.ds(16, 16)] = zf
            prow[pl.ds(32, 16)] = zf
            nch = (n + 15) // 16

            def node_loop(i, _):
                node = st + i
                vals = hv[pl.ds(node * C, 16)]
                kvec = jnp.full((16,), vals[3], _F)
                ivec = jnp.full((16,), i, _I)

                def chunkf(m, cv):
                    pos = iota + m * 16
                    keys = plsc.load_gather(hv, [(st + pos) * C + 3])
                    beats = jnp.logical_or(
                        keys > kvec,
                        jnp.logical_and(keys == kvec, pos < ivec))
                    cond = jnp.logical_and(pos < n, beats)
                    return cv + jnp.where(cond, one, zi)

                cv = lax.fori_loop(0, nch, chunkf, zi)
                cnt = jnp.sum(cv)

                @pl.when(cnt < KP)
                def _():
                    plsc.store_scatter(prow, [cnt * C + iota], vals,
                                       mask=wmask)
                return 0

            lax.fori_loop(0, n, node_loop, 0)
            pltpu.sync_copy(prow, pool_hbm.at[g])


def _make_sc_pool():
    return pl.kernel(
        _sc_pool_body,
        out_type=jax.ShapeDtypeStruct((G, KP * C), _F),
        mesh=_mesh(),
        compiler_params=pltpu.CompilerParams(needs_layout_passes=False),
        scratch_types=[
            pltpu.VMEM((NF,), _F),
            pltpu.VMEM((G + 16,), _I),
            pltpu.VMEM((G + 16,), _I),
            pltpu.VMEM((KP * C,), _F),
        ])


def _tc_misc_body(deg_ref, batf_ref, dinv_ref, counts_ref, starts_ref):
    d = deg_ref[...]
    dinv_ref[...] = jnp.where(d > 0.0, 1.0 / jnp.sqrt(d), 0.0)
    bat = batf_ref[...]
    gid = lax.broadcasted_iota(_I, (G, NP), 0)
    eq = (bat == gid)
    cnts = jnp.sum(jnp.where(eq, 1.0, 0.0), axis=1, keepdims=True)
    ltm = (lax.broadcasted_iota(_I, (G, G), 0)
           > lax.broadcasted_iota(_I, (G, G), 1)).astype(_F)
    sts = jnp.dot(ltm, cnts, preferred_element_type=_F)
    counts_ref[...] = cnts.astype(_I)
    starts_ref[...] = sts.astype(_I)


def _tc_proj_body(x_ref, h1_ref, h2_ref, h3_ref, w0_ref, w1_ref, w2_ref,
                  w3_ref, b_ref, o_ref):
    out = jnp.dot(x_ref[...], w0_ref[...], preferred_element_type=_F)
    out = out + jnp.dot(h1_ref[...], w1_ref[...], preferred_element_type=_F)
    out = out + jnp.dot(h2_ref[...], w2_ref[...], preferred_element_type=_F)
    out = out + jnp.dot(h3_ref[...], w3_ref[...], preferred_element_type=_F)
    o_ref[...] = jnp.maximum(out + b_ref[...], 0.0)


def _tc_head_body(p_ref, w_ref, cb_ref, fw_ref, fb_ref, o_ref):
    c1 = jnp.maximum(
        jnp.dot(p_ref[...], w_ref[...], preferred_element_type=_F)
        + cb_ref[...], 0.0)
    o_ref[...] = jnp.dot(c1, fw_ref[...], preferred_element_type=_F) \
        + fb_ref[...]


@functools.lru_cache(maxsize=None)
def _calls():
    return dict(
        sc_bin=_make_sc_bin(),
        sc_normb=_make_sc_normb(),
        sc_hop128=_make_sc_hop128(),
        sc_hop4=_make_sc_hop4(),
        sc_pool=_make_sc_pool(),
        tc_misc=pl.pallas_call(
            _tc_misc_body,
            out_shape=(jax.ShapeDtypeStruct((1, NP), _F),
                       jax.ShapeDtypeStruct((G, 1), _I),
                       jax.ShapeDtypeStruct((G, 1), _I))),
        tc_proj1=pl.pallas_call(
            _tc_proj_body,
            out_shape=jax.ShapeDtypeStruct((NP, C), _F)),
        tc_proj2=pl.pallas_call(
            _tc_proj_body,
            out_shape=jax.ShapeDtypeStruct((NP, C), _F)),
        tc_head=pl.pallas_call(
            _tc_head_body,
            out_shape=jax.ShapeDtypeStruct((G, 2), _F)),
    )


def kernel(x, edge_index, batch, edge_weight, conv1_W, conv1_b, conv2_W,
           conv2_b, conv1d_W, conv1d_b, fc_W, fc_b):
    k = _calls()
    row = edge_index[0].astype(_I)
    col = edge_index[1].astype(_I)
    ewt = edge_weight.astype(_F).reshape(-1)
    xp = jnp.pad(x.astype(_F), ((0, NP - N), (0, 0)))
    batf = jnp.pad(batch.astype(_I), (0, NP - N),
                   constant_values=G).reshape(1, NP)

    brow, bcol, bew, deg = k["sc_bin"](row, col, ewt)
    dinv2, counts2, starts2 = k["tc_misc"](deg.reshape(1, NP), batf)
    dinv = dinv2.reshape(NP)
    counts = counts2.reshape(G)
    starts = starts2.reshape(G)
    bnorm = k["sc_normb"](brow, bcol, bew, dinv)

    h1 = k["sc_hop128"](xp, brow, bnorm, bcol)
    h2 = k["sc_hop128"](h1, brow, bnorm, bcol)
    h3 = k["sc_hop128"](h2, brow, bnorm, bcol)
    w1 = conv1_W.astype(_F)
    g1 = k["tc_proj1"](xp, h1, h2, h3, w1[0].T, w1[1].T, w1[2].T, w1[3].T,
                       conv1_b.astype(_F).reshape(1, C))

    g1f = g1.reshape(NF)
    u1 = k["sc_hop4"](g1f, brow, bnorm, bcol)
    u2 = k["sc_hop4"](u1, brow, bnorm, bcol)
    u3 = k["sc_hop4"](u2, brow, bnorm, bcol)
    w2 = conv2_W.astype(_F)
    h2o = k["tc_proj2"](g1, u1.reshape(NP, C), u2.reshape(NP, C),
                        u3.reshape(NP, C), w2[0].T, w2[1].T, w2[2].T,
                        w2[3].T, conv2_b.astype(_F).reshape(1, C))

    pool = k["sc_pool"](h2o.reshape(NF), starts, counts)

    wflat_t = conv1d_W.astype(_F).transpose(0, 2, 1).reshape(C, KP * C).T
    out = k["tc_head"](pool, wflat_t,
                       conv1d_b.astype(_F).reshape(1, C),
                       fc_W.astype(_F).T, fc_b.astype(_F).reshape(1, 2))
    return out


# speedup vs baseline: 2.0049x; 1.0055x over previous
import functools

import jax
import jax.numpy as jnp
from jax import lax
from jax.experimental import pallas as pl
from jax.experimental.pallas import tpu as pltpu
from jax.experimental.pallas import tpu_sc as plsc

N = 10000
E = 322560
G = 100
KP = 12
C = 4
D = 128
NEW = 4032
NC = 2
NS = 16
NW = NC * NS
NP = 10240
NF = NP * C
RT = NP // NW
EMAX = 12288
CHB = 5040
CHE = 128

_F = jnp.float32
_I = jnp.int32


def _mesh():
    return plsc.VectorSubcoreMesh(
        core_axis_name="c", subcore_axis_name="s",
        num_cores=NC, num_subcores=NS)


def _wid():
    return lax.axis_index("s") * NC + lax.axis_index("c")


def _zero_ref(ref, nwords):
    z = jnp.zeros((16,), _F)

    def body(i, _):
        ref[pl.ds(i * 16, 16)] = z
        return 0
    lax.fori_loop(0, nwords // 16, body, 0)


def _sc_bin_body(row_hbm, col_hbm, ewt_hbm, brow_hbm, bcol_hbm, bew_hbm,
                 deg_hbm, rbuf, cbuf, ewt, lrow, lcol, lew, dacc):
    wid = _wid()
    base_n = wid * RT
    iota = lax.iota(_I, 16)
    pltpu.sync_copy(ewt_hbm, ewt)
    _zero_ref(dacc, RT)
    _zero_ref(lew, EMAX)
    zi = jnp.zeros((16,), _I)

    def zl(i, _):
        lrow[pl.ds(i * 16, 16)] = zi
        lcol[pl.ds(i * 16, 16)] = zi
        return 0
    lax.fori_loop(0, EMAX // 16, zl, 0)

    def chunk(k, off):
        eb = k * CHB
        pltpu.sync_copy(row_hbm.at[pl.ds(eb, CHB)], rbuf)
        pltpu.sync_copy(col_hbm.at[pl.ds(eb, CHB)], cbuf)

        def grp(g, off):
            sl = pl.ds(g * 16, 16)
            r16 = rbuf[sl]
            c16 = cbuf[sl]
            e16 = eb + g * 16 + iota
            m16 = e16 % NEW
            ewv = plsc.load_gather(ewt, [m16])
            loc = c16 - base_n
            sel = jnp.logical_and(loc >= 0, loc < RT)
            locc = jnp.minimum(jnp.maximum(loc, 0), RT - 1)
            cnt = plsc.all_reduce_population_count(sel)[0]

            @pl.when(off <= EMAX - 16)
            def _():
                plsc.addupdate_scatter(dacc, [locc], ewv, mask=sel)
                plsc.store_compressed(lrow.at[pl.ds(off, 16)], r16, mask=sel)
                plsc.store_compressed(lcol.at[pl.ds(off, 16)], c16, mask=sel)
                plsc.store_compressed(lew.at[pl.ds(off, 16)], ewv, mask=sel)
            return off + cnt
        return lax.fori_loop(0, CHB // 16, grp, off)

    lax.fori_loop(0, E // CHB, chunk, jnp.int32(0))
    base_e = wid * EMAX
    pltpu.sync_copy(lrow, brow_hbm.at[pl.ds(base_e, EMAX)])
    pltpu.sync_copy(lcol, bcol_hbm.at[pl.ds(base_e, EMAX)])
    pltpu.sync_copy(lew, bew_hbm.at[pl.ds(base_e, EMAX)])
    pltpu.sync_copy(dacc, deg_hbm.at[pl.ds(base_n, RT)])


def _make_sc_bin():
    return pl.kernel(
        _sc_bin_body,
        out_type=(jax.ShapeDtypeStruct((NW * EMAX,), _I),
                  jax.ShapeDtypeStruct((NW * EMAX,), _I),
                  jax.ShapeDtypeStruct((NW * EMAX,), _F),
                  jax.ShapeDtypeStruct((NP,), _F)),
        mesh=_mesh(),
        compiler_params=pltpu.CompilerParams(needs_layout_passes=False),
        scratch_types=[
            pltpu.VMEM((CHB,), _I),
            pltpu.VMEM((CHB,), _I),
            pltpu.VMEM((NEW,), _F),
            pltpu.VMEM((EMAX,), _I),
            pltpu.VMEM((EMAX,), _I),
            pltpu.VMEM((EMAX,), _F),
            pltpu.VMEM((RT,), _F),
        ])


def _sc_normb_body(brow_hbm, bcol_hbm, bew_hbm, dinv_hbm, bnorm_hbm,
                   dv, rbuf, cbuf, wbuf, nbuf):
    wid = _wid()
    base_e = wid * EMAX
    pltpu.sync_copy(dinv_hbm, dv)
    pltpu.sync_copy(brow_hbm.at[pl.ds(base_e, EMAX)], rbuf)
    pltpu.sync_copy(bcol_hbm.at[pl.ds(base_e, EMAX)], cbuf)
    pltpu.sync_copy(bew_hbm.at[pl.ds(base_e, EMAX)], wbuf)

    def body(g, _):
        sl = pl.ds(g * 16, 16)
        a = plsc.load_gather(dv, [rbuf[sl]])
        b = plsc.load_gather(dv, [cbuf[sl]])
        nbuf[sl] = a * wbuf[sl] * b
        return 0
    lax.fori_loop(0, EMAX // 16, body, 0)
    pltpu.sync_copy(nbuf, bnorm_hbm.at[pl.ds(base_e, EMAX)])


def _make_sc_normb():
    return pl.kernel(
        _sc_normb_body,
        out_type=jax.ShapeDtypeStruct((NW * EMAX,), _F),
        mesh=_mesh(),
        compiler_params=pltpu.CompilerParams(needs_layout_passes=False),
        scratch_types=[
            pltpu.VMEM((NP,), _F),
            pltpu.VMEM((EMAX,), _I),
            pltpu.VMEM((EMAX,), _I),
            pltpu.VMEM((EMAX,), _F),
            pltpu.VMEM((EMAX,), _F),
        ])


def _sc_hop128_body(zh_hbm, brow_hbm, bnorm_hbm, bcol_hbm, out_hbm,
                    acc, gbuf, rbuf, cbuf, nbuf, sem):
    wid = _wid()
    base_n = wid * RT
    base_e = wid * EMAX
    pltpu.sync_copy(brow_hbm.at[pl.ds(base_e, EMAX)], rbuf)
    pltpu.sync_copy(bcol_hbm.at[pl.ds(base_e, EMAX)], cbuf)
    pltpu.sync_copy(bnorm_hbm.at[pl.ds(base_e, EMAX)], nbuf)

    def zrow(i, _):
        for c in range(D // 16):
            acc[i, pl.ds(c * 16, 16)] = jnp.zeros((16,), _F)
        return 0
    lax.fori_loop(0, RT, zrow, 0)

    NB = 2

    def chunk(k, _):
        cb = k * NB * CHE
        descs = []
        for q in range(NB):
            descs.append(pltpu.async_copy(
                zh_hbm.at[rbuf.at[pl.ds(cb + q * CHE, CHE)]],
                gbuf.at[pl.ds(q * CHE, CHE)], sem))
        for d in descs:
            d.wait()

        def edge(e, _):
            nrm = nbuf[pl.ds(cb + e, 16)][0]
            loc = cbuf[pl.ds(cb + e, 16)][0] - base_n
            loc = jnp.minimum(jnp.maximum(loc, 0), RT - 1)
            nv = jnp.full((16,), nrm, _F)
            for c in range(D // 16):
                s = pl.ds(c * 16, 16)
                acc[loc, s] = acc[loc, s] + gbuf[e, s] * nv
            return 0
        lax.fori_loop(0, NB * CHE, edge, 0)
        return 0
    lax.fori_loop(0, EMAX // (NB * CHE), chunk, 0)
    pltpu.sync_copy(acc, out_hbm.at[pl.ds(base_n, RT)])


def _make_sc_hop128():
    return pl.kernel(
        _sc_hop128_body,
        out_type=jax.ShapeDtypeStruct((NP, D), _F),
        mesh=_mesh(),
        compiler_params=pltpu.CompilerParams(needs_layout_passes=False),
        scratch_types=[
            pltpu.VMEM((RT, D), _F),
            pltpu.VMEM((2 * CHE, D), _F),
            pltpu.VMEM((EMAX,), _I),
            pltpu.VMEM((EMAX,), _I),
            pltpu.VMEM((EMAX,), _F),
            pltpu.SemaphoreType.DMA,
        ])


def _sc_hop4_body(zh_hbm, brow_hbm, bnorm_hbm, bcol_hbm, out_hbm,
                  zv, acc, rbuf, cbuf, nbuf):
    wid = _wid()
    base_n = wid * RT
    base_e = wid * EMAX
    pltpu.sync_copy(zh_hbm, zv)
    pltpu.sync_copy(brow_hbm.at[pl.ds(base_e, EMAX)], rbuf)
    pltpu.sync_copy(bcol_hbm.at[pl.ds(base_e, EMAX)], cbuf)
    pltpu.sync_copy(bnorm_hbm.at[pl.ds(base_e, EMAX)], nbuf)
    _zero_ref(acc, RT * C)

    def body(g, _):
        sl = pl.ds(g * 16, 16)
        r16 = rbuf[sl] * C
        n16 = nbuf[sl]
        loc = cbuf[sl] - base_n
        loc = jnp.minimum(jnp.maximum(loc, 0), RT - 1) * C
        for c in range(C):
            v = plsc.load_gather(zv, [r16 + c]) * n16
            plsc.addupdate_scatter(acc, [loc + c], v)
        return 0
    lax.fori_loop(0, EMAX // 16, body, 0)
    pltpu.sync_copy(acc, out_hbm.at[pl.ds(base_n * C, RT * C)])


def _make_sc_hop4():
    return pl.kernel(
        _sc_hop4_body,
        out_type=jax.ShapeDtypeStruct((NF,), _F),
        mesh=_mesh(),
        compiler_params=pltpu.CompilerParams(needs_layout_passes=False),
        scratch_types=[
            pltpu.VMEM((NF,), _F),
            pltpu.VMEM((RT * C,), _F),
            pltpu.VMEM((EMAX,), _I),
            pltpu.VMEM((EMAX,), _I),
            pltpu.VMEM((EMAX,), _F),
        ])


def _sc_pool_body(h2_hbm, starts_hbm, counts_hbm, pool_hbm, hv, stb, ctb,
                  prow):
    wid = _wid()
    pltpu.sync_copy(h2_hbm, hv)
    pltpu.sync_copy(starts_hbm, stb.at[pl.ds(0, G)])
    pltpu.sync_copy(counts_hbm, ctb.at[pl.ds(0, G)])
    iota = lax.iota(_I, 16)
    zf = jnp.zeros((16,), _F)
    one = jnp.ones((16,), _I)
    zi = jnp.zeros((16,), _I)
    wmask = iota < C

    for j in range((G + NW - 1) // NW):
        g = wid + NW * j

        @pl.when(g < G)
        def _():
            st = stb[pl.ds(g, 16)][0]
            n = ctb[pl.ds(g, 16)][0]
            prow[pl.ds(0, 16)] = zf
            prow[pl.ds(16, 16)] = zf
            prow[pl.ds(32, 16)] = zf
            nch = (n + 15) // 16

            def node_loop(i, _):
                node = st + i
                vals = hv[pl.ds(node * C, 16)]
                kvec = jnp.full((16,), vals[3], _F)
                ivec = jnp.full((16,), i, _I)

                def chunkf(m, cv):
                    pos = iota + m * 16
                    keys = plsc.load_gather(hv, [(st + pos) * C + 3])
                    beats = jnp.logical_or(
                        keys > kvec,
                        jnp.logical_and(keys == kvec, pos < ivec))
                    cond = jnp.logical_and(pos < n, beats)
                    return cv + jnp.where(cond, one, zi)

                cv = lax.fori_loop(0, nch, chunkf, zi)
                cnt = jnp.sum(cv)

                @pl.when(cnt < KP)
                def _():
                    plsc.store_scatter(prow, [cnt * C + iota], vals,
                                       mask=wmask)
                return 0

            lax.fori_loop(0, n, node_loop, 0)
            pltpu.sync_copy(prow, pool_hbm.at[g])


def _make_sc_pool():
    return pl.kernel(
        _sc_pool_body,
        out_type=jax.ShapeDtypeStruct((G, KP * C), _F),
        mesh=_mesh(),
        compiler_params=pltpu.CompilerParams(needs_layout_passes=False),
        scratch_types=[
            pltpu.VMEM((NF,), _F),
            pltpu.VMEM((G + 16,), _I),
            pltpu.VMEM((G + 16,), _I),
            pltpu.VMEM((KP * C,), _F),
        ])


def _tc_misc_body(deg_ref, batf_ref, dinv_ref, counts_ref, starts_ref):
    d = deg_ref[...]
    dinv_ref[...] = jnp.where(d > 0.0, 1.0 / jnp.sqrt(d), 0.0)
    bat = batf_ref[...]
    gid = lax.broadcasted_iota(_I, (G, NP), 0)
    eq = (bat == gid)
    cnts = jnp.sum(jnp.where(eq, 1.0, 0.0), axis=1, keepdims=True)
    ltm = (lax.broadcasted_iota(_I, (G, G), 0)
           > lax.broadcasted_iota(_I, (G, G), 1)).astype(_F)
    sts = jnp.dot(ltm, cnts, preferred_element_type=_F)
    counts_ref[...] = cnts.astype(_I)
    starts_ref[...] = sts.astype(_I)


def _tc_proj_body(x_ref, h1_ref, h2_ref, h3_ref, w0_ref, w1_ref, w2_ref,
                  w3_ref, b_ref, o_ref):
    out = jnp.dot(x_ref[...], w0_ref[...], preferred_element_type=_F)
    out = out + jnp.dot(h1_ref[...], w1_ref[...], preferred_element_type=_F)
    out = out + jnp.dot(h2_ref[...], w2_ref[...], preferred_element_type=_F)
    out = out + jnp.dot(h3_ref[...], w3_ref[...], preferred_element_type=_F)
    o_ref[...] = jnp.maximum(out + b_ref[...], 0.0)


def _tc_head_body(p_ref, w_ref, cb_ref, fw_ref, fb_ref, o_ref):
    c1 = jnp.maximum(
        jnp.dot(p_ref[...], w_ref[...], preferred_element_type=_F)
        + cb_ref[...], 0.0)
    o_ref[...] = jnp.dot(c1, fw_ref[...], preferred_element_type=_F) \
        + fb_ref[...]


@functools.lru_cache(maxsize=None)
def _calls():
    return dict(
        sc_bin=_make_sc_bin(),
        sc_normb=_make_sc_normb(),
        sc_hop128=_make_sc_hop128(),
        sc_hop4=_make_sc_hop4(),
        sc_pool=_make_sc_pool(),
        tc_misc=pl.pallas_call(
            _tc_misc_body,
            out_shape=(jax.ShapeDtypeStruct((1, NP), _F),
                       jax.ShapeDtypeStruct((G, 1), _I),
                       jax.ShapeDtypeStruct((G, 1), _I))),
        tc_proj1=pl.pallas_call(
            _tc_proj_body,
            out_shape=jax.ShapeDtypeStruct((NP, C), _F)),
        tc_proj2=pl.pallas_call(
            _tc_proj_body,
            out_shape=jax.ShapeDtypeStruct((NP, C), _F)),
        tc_head=pl.pallas_call(
            _tc_head_body,
            out_shape=jax.ShapeDtypeStruct((G, 2), _F)),
    )


def kernel(x, edge_index, batch, edge_weight, conv1_W, conv1_b, conv2_W,
           conv2_b, conv1d_W, conv1d_b, fc_W, fc_b):
    k = _calls()
    row = edge_index[0].astype(_I)
    col = edge_index[1].astype(_I)
    ewt = edge_weight.astype(_F).reshape(-1)
    xp = jnp.pad(x.astype(_F), ((0, NP - N), (0, 0)))
    batf = jnp.pad(batch.astype(_I), (0, NP - N),
                   constant_values=G).reshape(1, NP)

    brow, bcol, bew, deg = k["sc_bin"](row, col, ewt)
    dinv2, counts2, starts2 = k["tc_misc"](deg.reshape(1, NP), batf)
    dinv = dinv2.reshape(NP)
    counts = counts2.reshape(G)
    starts = starts2.reshape(G)
    bnorm = k["sc_normb"](brow, bcol, bew, dinv)

    h1 = k["sc_hop128"](xp, brow, bnorm, bcol)
    h2 = k["sc_hop128"](h1, brow, bnorm, bcol)
    h3 = k["sc_hop128"](h2, brow, bnorm, bcol)
    w1 = conv1_W.astype(_F)
    g1 = k["tc_proj1"](xp, h1, h2, h3, w1[0].T, w1[1].T, w1[2].T, w1[3].T,
                       conv1_b.astype(_F).reshape(1, C))

    g1f = g1.reshape(NF)
    u1 = k["sc_hop4"](g1f, brow, bnorm, bcol)
    u2 = k["sc_hop4"](u1, brow, bnorm, bcol)
    u3 = k["sc_hop4"](u2, brow, bnorm, bcol)
    w2 = conv2_W.astype(_F)
    h2o = k["tc_proj2"](g1, u1.reshape(NP, C), u2.reshape(NP, C),
                        u3.reshape(NP, C), w2[0].T, w2[1].T, w2[2].T,
                        w2[3].T, conv2_b.astype(_F).reshape(1, C))

    pool = k["sc_pool"](h2o.reshape(NF), starts, counts)

    wflat_t = conv1d_W.astype(_F).transpose(0, 2, 1).reshape(C, KP * C).T
    out = k["tc_head"](pool, wflat_t,
                       conv1d_b.astype(_F).reshape(1, C),
                       fc_W.astype(_F).T, fc_b.astype(_F).reshape(1, 2))
    return out
